# baseline (speedup 1.0000x reference)
import functools

import jax
import jax.numpy as jnp
from jax import lax
from jax.experimental import pallas as pl
from jax.experimental.pallas import tpu as pltpu
from jax.experimental.pallas import tpu_sc as plsc

D = 256
C = 4
N = 4096
L = 16
VOL = D * D * D
GROWS = C * VOL // L

_GATHER_DNUMS = jax.lax.GatherDimensionNumbers(
    offset_dims=(), collapsed_slice_dims=(0,), start_index_map=(0,))


def _lane_pick(vec, lane_vec):
    return lax.gather(vec, lane_vec[:, None], _GATHER_DNUMS, (1,),
                      mode=jax.lax.GatherScatterMode.PROMISE_IN_BOUNDS)


def _sc_partials(fix3, mov3, pts):
    info = plsc.get_sparse_core_info()
    nw = info.num_cores * info.num_subcores
    ppw = N // nw
    groups = ppw // L
    mesh = plsc.VectorSubcoreMesh(core_axis_name="c", subcore_axis_name="s")

    @functools.partial(
        pl.kernel,
        out_type=jax.ShapeDtypeStruct((nw, L), jnp.float32),
        mesh=mesh,
        compiler_params=pltpu.CompilerParams(use_tc_tiling_on_sc=False),
        scratch_types=[
            pltpu.VMEM((6 * ppw,), jnp.int32),
            pltpu.VMEM((C, ppw), jnp.int32),
            pltpu.VMEM((C, ppw), jnp.int32),
            pltpu.VMEM((C * ppw, L), jnp.float32),
            pltpu.VMEM((C * ppw, L), jnp.float32),
            pltpu.VMEM((L,), jnp.float32),
            pltpu.SemaphoreType.DMA,
        ],
    )
    def k(fix_rows, mov_rows, pts_hbm, out_hbm,
          pts_v, rowf_v, rown_v, f_v, n_v, acc_v, sem):
        wid = lax.axis_index("s") * info.num_cores + lax.axis_index("c")
        base = wid * ppw
        for r in range(6):
            pltpu.sync_copy(pts_hbm.at[pl.ds(r * N + base, ppw)],
                            pts_v.at[pl.ds(r * ppw, ppw)])

        for g in range(groups):
            for seg, row_ref in ((0, rowf_v), (3, rown_v)):
                x = pts_v[pl.ds((seg + 0) * ppw + g * L, L)]
                y = pts_v[pl.ds((seg + 1) * ppw + g * L, L)]
                z = pts_v[pl.ds((seg + 2) * ppw + g * L, L)]
                row = lax.shift_right_logical(x * (D * D) + y * D + z, 4)
                for c in range(C):
                    row_ref[c, pl.ds(g * L, L)] = row + c * (VOL // L)

        copies = []
        for c in range(C):
            copies.append(pltpu.async_copy(
                fix_rows.at[rowf_v.at[c]], f_v.at[pl.ds(c * ppw, ppw)], sem))
            copies.append(pltpu.async_copy(
                mov_rows.at[rown_v.at[c]], n_v.at[pl.ds(c * ppw, ppw)], sem))
        for cp in copies:
            cp.wait()

        lanes = lax.iota(jnp.int32, L)

        def body(g, acc):
            zf_vec = pts_v[pl.ds(2 * ppw + g * L, L)]
            zn_vec = pts_v[pl.ds(5 * ppw + g * L, L)]
            bf_vec = lax.bitwise_and(zf_vec, L - 1)
            bn_vec = lax.bitwise_and(zn_vec, L - 1)
            for i in range(L):
                bfs = jnp.full((L,), bf_vec[i], jnp.int32)
                bns = jnp.full((L,), bn_vec[i], jnp.int32)
                dsum = jnp.zeros((L,), jnp.float32)
                for c in range(C):
                    fsp = _lane_pick(f_v[c * ppw + g * L + i, :], bfs)
                    msp = _lane_pick(n_v[c * ppw + g * L + i, :], bns)
                    d = fsp - msp
                    dsum = dsum + d * d
                acc = acc + jnp.where(lanes == i, dsum, 0.0)
            return acc

        acc = lax.fori_loop(0, groups, body, jnp.zeros((L,), jnp.float32))
        acc_v[...] = acc
        pltpu.sync_copy(acc_v, out_hbm.at[wid])

    return k(fix3, mov3, pts)


def _finalize_kernel(p_ref, o_ref):
    s = jnp.sum(p_ref[...])
    loss = (0.01 * N - s) * (10000.0 / (2.0 * N))
    o_ref[...] = jnp.broadcast_to(loss, (1, 1))


def kernel(fix_image_feature, moving_image_feature, fixed_points,
           positive_points, negative_points):
    del positive_points
    pts = jnp.concatenate(
        [fixed_points.T.reshape(-1), negative_points.T.reshape(-1)])
    partials = _sc_partials(fix_image_feature.reshape(GROWS, L),
                            moving_image_feature.reshape(GROWS, L),
                            pts)
    loss = pl.pallas_call(
        _finalize_kernel,
        out_shape=jax.ShapeDtypeStruct((1, 1), jnp.float32),
    )(partials)
    return loss[0, 0]
